# baseline (speedup 1.0000x reference)
import functools

import jax
import jax.numpy as jnp
from jax import lax
from jax.experimental import pallas as pl
from jax.experimental.pallas import tpu as pltpu
from jax.experimental.pallas import tpu_sc as plsc

_E = 64
_LANES = 16
_NW = 32
_CH = 256


def _tc_probs_block(fac_ref, x_ref, w_ref, probs_ref):
    f = fac_ref[0, 0]
    x = x_ref[...] * f
    w = w_ref[...]
    logits = jax.lax.dot_general(
        x, w, (((1,), (1,)), ((), ())),
        preferred_element_type=jnp.float32)
    m = jnp.max(logits, axis=-1, keepdims=True)
    ex = jnp.exp(logits - m)
    z = jnp.sum(ex, axis=-1, keepdims=True)
    probs_ref[...] = ex / z


def _tc_probs(xf, w, factor, block_t=4096):
    n_tok, d = xf.shape
    e = w.shape[0]
    return pl.pallas_call(
        _tc_probs_block,
        grid=(n_tok // block_t,),
        in_specs=[
            pl.BlockSpec((1, 1), lambda i: (0, 0)),
            pl.BlockSpec((block_t, d), lambda i: (i, 0)),
            pl.BlockSpec((e, d), lambda i: (0, 0)),
        ],
        out_specs=pl.BlockSpec((block_t, e), lambda i: (i, 0)),
        out_shape=jax.ShapeDtypeStruct((n_tok, e), jnp.float32),
    )(factor, xf, w)


def _lane_perm(v, idx):
    return lax.gather(
        v, idx[:, None],
        dimension_numbers=lax.GatherDimensionNumbers(
            offset_dims=(), collapsed_slice_dims=(0,), start_index_map=(0,)),
        slice_sizes=(1,), mode=lax.GatherScatterMode.PROMISE_IN_BOUNDS)


def _top2_keys(k0, k1, k2, k3, perms):
    aw = jnp.maximum(k0, k1)
    al = jnp.minimum(k0, k1)
    bw = jnp.maximum(k2, k3)
    bl = jnp.minimum(k2, k3)
    m1 = jnp.maximum(aw, bw)
    m2 = jnp.maximum(jnp.minimum(aw, bw), jnp.maximum(al, bl))
    for p in perms:
        o1 = _lane_perm(m1, p)
        o2 = _lane_perm(m2, p)
        t = jnp.minimum(m1, o1)
        m1 = jnp.maximum(m1, o1)
        m2 = jnp.maximum(jnp.maximum(m2, o2), t)
    return m1, m2


def _sc_route_body(probs_hbm, disp_hbm, idx_hbm, pin, pdisp, pidx):
    nc = 2
    wid = lax.axis_index("s") * nc + lax.axis_index("c")
    n_tok = idx_hbm.shape[0] // 2
    tw = n_tok // _NW
    base = wid * tw

    lanes = lax.iota(jnp.int32, _LANES)
    half_lo = lax.shift_right_logical(lanes, 1)
    half_hi = half_lo + 8
    even_lane = lax.bitwise_and(lanes, 1) == 0
    perms = [lanes ^ s for s in (1, 2, 4, 8)]
    low_mask = jnp.int32(-64)
    tag0 = 63 - lanes
    tag1 = 47 - lanes
    tag2 = 31 - lanes
    tag3 = 15 - lanes

    def load_key(ref, off, tag):
        b = lax.bitcast_convert_type(ref[pl.ds(off, _LANES)], jnp.int32)
        return lax.bitcast_convert_type(
            lax.bitwise_or(lax.bitwise_and(b, low_mask), tag), jnp.float32)

    for c in range(tw // _CH):
        t0 = base + c * _CH
        pltpu.sync_copy(probs_hbm.at[pl.ds(t0 * _E, _CH * _E)], pin)

        def group_body(g, carry):
            i1acc, i2acc = carry
            gbase = g * (_LANES * _E)
            for j in range(_LANES):
                off = gbase + j * _E
                k0 = load_key(pin, off, tag0)
                k1 = load_key(pin, off + 16, tag1)
                k2 = load_key(pin, off + 32, tag2)
                k3 = load_key(pin, off + 48, tag3)
                g1k, g2k = _top2_keys(k0, k1, k2, k3, perms)
                g1b = lax.bitcast_convert_type(g1k, jnp.int32)
                g2b = lax.bitcast_convert_type(g2k, jnp.int32)
                i1 = 63 - lax.bitwise_and(g1b, 63)
                i2 = 63 - lax.bitwise_and(g2b, 63)
                g1v = lax.bitcast_convert_type(
                    lax.bitwise_and(g1b, low_mask), jnp.float32)
                g2v = lax.bitcast_convert_type(
                    lax.bitwise_and(g2b, low_mask), jnp.float32)
                r = 1.0 / (g1v + g2v)
                w1 = g1v * r
                w2 = g2v * r
                zero = jnp.zeros((_LANES,), jnp.float32)
                d0 = jnp.where(k0 == g1k, w1, jnp.where(k0 == g2k, w2, zero))
                d1 = jnp.where(k1 == g1k, w1, jnp.where(k1 == g2k, w2, zero))
                d2 = jnp.where(k2 == g1k, w1, jnp.where(k2 == g2k, w2, zero))
                d3 = jnp.where(k3 == g1k, w1, jnp.where(k3 == g2k, w2, zero))
                pdisp[pl.ds(off, _LANES)] = d0
                pdisp[pl.ds(off + 16, _LANES)] = d1
                pdisp[pl.ds(off + 32, _LANES)] = d2
                pdisp[pl.ds(off + 48, _LANES)] = d3
                i1acc = jnp.where(lanes == j, i1, i1acc)
                i2acc = jnp.where(lanes == j, i2, i2acc)
            va = jnp.where(even_lane, _lane_perm(i1acc, half_lo),
                           _lane_perm(i2acc, half_lo))
            vb = jnp.where(even_lane, _lane_perm(i1acc, half_hi),
                           _lane_perm(i2acc, half_hi))
            gofs = g * (_LANES * 2)
            pidx[pl.ds(gofs, _LANES)] = va
            pidx[pl.ds(gofs + _LANES, _LANES)] = vb
            return (i1acc, i2acc)

        zi = jnp.zeros((_LANES,), jnp.int32)
        lax.fori_loop(0, _CH // _LANES, group_body, (zi, zi))
        pltpu.sync_copy(pdisp, disp_hbm.at[pl.ds(t0 * _E, _CH * _E)])
        pltpu.sync_copy(pidx, idx_hbm.at[pl.ds(t0 * 2, _CH * 2)])


def _sc_route(probs_flat, n_tok):
    mesh = plsc.VectorSubcoreMesh(core_axis_name="c", subcore_axis_name="s")
    f = pl.kernel(
        _sc_route_body,
        mesh=mesh,
        out_type=[
            jax.ShapeDtypeStruct((n_tok * _E,), jnp.float32),
            jax.ShapeDtypeStruct((n_tok * 2,), jnp.int32),
        ],
        scratch_types=[
            pltpu.VMEM((_CH * _E,), jnp.float32),
            pltpu.VMEM((_CH * _E,), jnp.float32),
            pltpu.VMEM((_CH * 2,), jnp.int32),
        ],
    )
    return f(probs_flat)


def kernel(x, scale_condition, W, scale_idx):
    b, s, d = x.shape
    e = W.shape[0]
    n_tok = b * s
    factor = (1.0 + scale_condition[scale_idx] * 0.1).reshape(1, 1)
    probs = _tc_probs(x.reshape(n_tok, d), W, factor)
    disp_flat, idx_flat = _sc_route(probs.reshape(n_tok * e), n_tok)
    return (disp_flat.reshape(b, s, e), probs.reshape(b, s, e),
            idx_flat.reshape(b, s, 2))


# speedup vs baseline: 3.3427x; 3.3427x over previous
import functools

import jax
import jax.numpy as jnp
from jax import lax
from jax.experimental import pallas as pl
from jax.experimental.pallas import tpu as pltpu


def _router_block(fac_ref, x_ref, w_ref, disp_ref, probs_ref, idx_ref):
    f = fac_ref[0, 0]
    x = x_ref[0] * f
    w = w_ref[...]
    lt = jax.lax.dot_general(
        w, x, (((1,), (1,)), ((), ())),
        preferred_element_type=jnp.float32)
    m = jnp.max(lt, axis=0, keepdims=True)
    ex = jnp.exp(lt - m)
    z = jnp.sum(ex, axis=0, keepdims=True)
    probs = ex / z
    probs_ref[0] = probs

    bits = lax.bitcast_convert_type(probs, jnp.int32)
    tag = 63 - lax.broadcasted_iota(jnp.int32, probs.shape, 0)
    keys = lax.bitcast_convert_type(
        lax.bitwise_or(lax.bitwise_and(bits, jnp.int32(-64)), tag),
        jnp.float32)
    m1 = jnp.max(keys, axis=0, keepdims=True)
    keys2 = jnp.where(keys == m1, -1.0, keys)
    m2 = jnp.max(keys2, axis=0, keepdims=True)
    b1 = lax.bitcast_convert_type(m1, jnp.int32)
    b2 = lax.bitcast_convert_type(m2, jnp.int32)
    i1 = 63 - lax.bitwise_and(b1, 63)
    i2 = 63 - lax.bitwise_and(b2, 63)
    g1 = lax.bitcast_convert_type(lax.bitwise_and(b1, jnp.int32(-64)),
                                  jnp.float32)
    g2 = lax.bitcast_convert_type(lax.bitwise_and(b2, jnp.int32(-64)),
                                  jnp.float32)
    r = 1.0 / (g1 + g2)
    w1 = g1 * r
    w2 = g2 * r
    disp_ref[0] = jnp.where(keys == m1, w1, jnp.where(keys == m2, w2, 0.0))
    idx_ref[0] = jnp.concatenate([i1, i2], axis=0)


def _route(x, w, factor, block_t=2048):
    b, s, d = x.shape
    e = w.shape[0]
    grid = (b, s // block_t)
    return pl.pallas_call(
        _router_block,
        grid=grid,
        in_specs=[
            pl.BlockSpec((1, 1), lambda i, j: (0, 0)),
            pl.BlockSpec((1, block_t, d), lambda i, j: (i, j, 0)),
            pl.BlockSpec((e, d), lambda i, j: (0, 0)),
        ],
        out_specs=[
            pl.BlockSpec((1, e, block_t), lambda i, j: (i, 0, j)),
            pl.BlockSpec((1, e, block_t), lambda i, j: (i, 0, j)),
            pl.BlockSpec((1, 2, block_t), lambda i, j: (i, 0, j)),
        ],
        out_shape=[
            jax.ShapeDtypeStruct((b, e, s), jnp.float32),
            jax.ShapeDtypeStruct((b, e, s), jnp.float32),
            jax.ShapeDtypeStruct((b, 2, s), jnp.int32),
        ],
    )(factor, x, w)


def kernel(x, scale_condition, W, scale_idx):
    factor = (1.0 + scale_condition[scale_idx] * 0.1).reshape(1, 1)
    disp_t, probs_t, idx_t = _route(x, W, factor)
    return (disp_t.transpose(0, 2, 1), probs_t.transpose(0, 2, 1),
            idx_t.transpose(0, 2, 1))


# speedup vs baseline: 3.5221x; 1.0537x over previous
import functools

import jax
import jax.numpy as jnp
from jax import lax
from jax.experimental import pallas as pl
from jax.experimental.pallas import tpu as pltpu


def _router_block(fac_ref, x_ref, w_ref, disp_ref, probs_ref, idx_ref):
    f = fac_ref[0, 0]
    x = x_ref[0] * f
    w = w_ref[...]
    lt = jax.lax.dot_general(
        w, x, (((1,), (1,)), ((), ())),
        preferred_element_type=jnp.float32)
    m = jnp.max(lt, axis=0, keepdims=True)
    ex = jnp.exp(lt - m)
    z = jnp.sum(ex, axis=0, keepdims=True)
    probs = ex / z
    probs_ref[0] = probs

    bits = lax.bitcast_convert_type(probs, jnp.int32)
    tag = 63 - lax.broadcasted_iota(jnp.int32, probs.shape, 0)
    keys = lax.bitcast_convert_type(
        lax.bitwise_or(lax.bitwise_and(bits, jnp.int32(-64)), tag),
        jnp.float32)
    m1 = jnp.max(keys, axis=0, keepdims=True)
    keys2 = jnp.where(keys == m1, -1.0, keys)
    m2 = jnp.max(keys2, axis=0, keepdims=True)
    b1 = lax.bitcast_convert_type(m1, jnp.int32)
    b2 = lax.bitcast_convert_type(m2, jnp.int32)
    i1 = 63 - lax.bitwise_and(b1, 63)
    i2 = 63 - lax.bitwise_and(b2, 63)
    g1 = lax.bitcast_convert_type(lax.bitwise_and(b1, jnp.int32(-64)),
                                  jnp.float32)
    g2 = lax.bitcast_convert_type(lax.bitwise_and(b2, jnp.int32(-64)),
                                  jnp.float32)
    r = 1.0 / (g1 + g2)
    w1 = g1 * r
    w2 = g2 * r
    disp_ref[0] = jnp.where(keys == m1, w1, jnp.where(keys == m2, w2, 0.0))
    idx_ref[0] = jnp.concatenate([i1, i2], axis=0)


def _route(x, w, factor, block_t=4096):
    b, s, d = x.shape
    e = w.shape[0]
    grid = (b, s // block_t)
    return pl.pallas_call(
        _router_block,
        grid=grid,
        in_specs=[
            pl.BlockSpec((1, 1), lambda i, j: (0, 0)),
            pl.BlockSpec((1, block_t, d), lambda i, j: (i, j, 0)),
            pl.BlockSpec((e, d), lambda i, j: (0, 0)),
        ],
        out_specs=[
            pl.BlockSpec((1, e, block_t), lambda i, j: (i, 0, j)),
            pl.BlockSpec((1, e, block_t), lambda i, j: (i, 0, j)),
            pl.BlockSpec((1, 2, block_t), lambda i, j: (i, 0, j)),
        ],
        out_shape=[
            jax.ShapeDtypeStruct((b, e, s), jnp.float32),
            jax.ShapeDtypeStruct((b, e, s), jnp.float32),
            jax.ShapeDtypeStruct((b, 2, s), jnp.int32),
        ],
    )(factor, x, w)


def kernel(x, scale_condition, W, scale_idx):
    factor = (1.0 + scale_condition[scale_idx] * 0.1).reshape(1, 1)
    disp_t, probs_t, idx_t = _route(x, W, factor)
    return (disp_t.transpose(0, 2, 1), probs_t.transpose(0, 2, 1),
            idx_t.transpose(0, 2, 1))
